# baseline (speedup 1.0000x reference)
import jax
import jax.numpy as jnp
from jax import lax
from jax.experimental import pallas as pl
from jax.experimental.pallas import tpu as pltpu
from jax.experimental.pallas import tpu_sc as plsc

B = 16384
H = 26
D = 32
N = B * H
V = 100000 * H
NC, NS, L = 2, 16, 16
NW = NC * NS
CHUNK = 26 * 64
CPW = N // (NW * CHUNK)
NPW = CPW * CHUNK
GRAN = 128
NG = CHUNK // GRAN

TBLK = 8192


def _tc_table_format(tbl_t):
    grid = (V + TBLK - 1) // TBLK

    def body(i_ref, o_ref):
        x = i_ref[...]
        o_ref[...] = x.reshape(D, TBLK // 4, 4).transpose(1, 2, 0).reshape(
            TBLK // 4, 128)

    return pl.pallas_call(
        body,
        grid=(grid,),
        in_specs=[pl.BlockSpec((D, TBLK), lambda i: (0, i))],
        out_specs=pl.BlockSpec((TBLK // 4, 128), lambda i: (i, 0)),
        out_shape=jax.ShapeDtypeStruct((V * D // 128, 128), jnp.float32),
    )(tbl_t)


def _sc_gather(flat_ids, pattern, table):
    mesh = plsc.VectorSubcoreMesh(core_axis_name="c", subcore_axis_name="s")

    @pl.kernel(
        mesh=mesh,
        out_type=jax.ShapeDtypeStruct((N, D), jnp.float32),
        scratch_types=[
            pltpu.VMEM((CHUNK,), jnp.int32),
            pltpu.VMEM((NPW,), jnp.int32),
            pltpu.VMEM((CHUNK, D), jnp.float32),
            pltpu.VMEM((CHUNK, D), jnp.float32),
            pltpu.SemaphoreType.DMA,
            pltpu.SemaphoreType.DMA,
        ],
        compiler_params=pltpu.CompilerParams(use_tc_tiling_on_sc=False),
    )
    def body(ids_hbm, pat_hbm, table_hbm, out_hbm,
             pat_v, idx_v, rows0_v, rows1_v, sem_g, sem_out):
        wid = lax.axis_index("s") * NC + lax.axis_index("c")
        wbase = wid * NPW
        pltpu.sync_copy(pat_hbm, pat_v)
        pltpu.sync_copy(ids_hbm.at[pl.ds(wbase, NPW)], idx_v)

        @pl.loop(0, CPW)
        def _add_chunk(c):
            @pl.loop(0, CHUNK // L)
            def _add(i):
                dst = pl.ds(c * CHUNK + i * L, L)
                idx_v[dst] = idx_v[dst] + pat_v[pl.ds(i * L, L)]

        def fire_drain_store(c, rows_v):
            handles = []
            for j in range(NG):
                sl = pl.ds(c * CHUNK + j * GRAN, GRAN)
                handles.append(pltpu.make_async_copy(
                    table_hbm.at[idx_v.at[sl]],
                    rows_v.at[pl.ds(j * GRAN, GRAN), :], sem_g))
            for h in handles:
                h.start()
            for h in handles:
                h.wait()
            st = pltpu.make_async_copy(
                rows_v, out_hbm.at[pl.ds(wbase + c * CHUNK, CHUNK)], sem_out)
            st.start()
            return st

        def wait_store(rows_v):
            pltpu.make_async_copy(
                out_hbm.at[pl.ds(wbase, CHUNK)], rows_v, sem_out).wait()

        fire_drain_store(0, rows0_v)
        fire_drain_store(1, rows1_v)

        @pl.loop(2, CPW, step=2)
        def _chunk(c):
            wait_store(rows0_v)
            fire_drain_store(c, rows0_v)
            wait_store(rows1_v)
            fire_drain_store(c + 1, rows1_v)

        wait_store(rows0_v)
        wait_store(rows1_v)

    return body(flat_ids, pattern, table)


def kernel(input_ids, offsets, table):
    tbl_lin = _tc_table_format(table.T)
    tbl_rm = tbl_lin.reshape(V, D)
    flat_ids = input_ids.reshape(N)
    pattern = jnp.tile(offsets, CHUNK // H)
    out = _sc_gather(flat_ids, pattern, tbl_rm)
    return out.reshape(B, H, D)


# speedup vs baseline: 9.9115x; 9.9115x over previous
import jax
import jax.numpy as jnp
from jax import lax
from jax.experimental import pallas as pl
from jax.experimental.pallas import tpu as pltpu
from jax.experimental.pallas import tpu_sc as plsc

B = 16384
H = 26
D = 32
N = B * H
V = 100000 * H
NC, NS, L = 2, 16, 16
NW = NC * NS
CHUNK = 26 * 64
CPW = N // (NW * CHUNK)
NPW = CPW * CHUNK
GRAN = 128
NG = CHUNK // GRAN

TBLK = 8192
TGRID = (V + TBLK - 1) // TBLK
VPAD = TGRID * TBLK


def _tc_table_format(tbl_t):
    def body(i_ref, o_ref):
        x = i_ref[...]
        for q in range(TBLK // 512):
            sq = jnp.concatenate(
                [x[:, q * 512 + a * 128:q * 512 + (a + 1) * 128]
                 for a in range(4)], axis=0)
            o_ref[pl.ds(q * 128, 128), :] = sq.T

    return pl.pallas_call(
        body,
        grid=(TGRID,),
        in_specs=[pl.BlockSpec((D, TBLK), lambda i: (0, i))],
        out_specs=pl.BlockSpec((TBLK // 4, 128), lambda i: (i, 0)),
        out_shape=jax.ShapeDtypeStruct((VPAD * D // 128, 128), jnp.float32),
    )(tbl_t)


def _sc_gather(flat_ids, pattern, table):
    mesh = plsc.VectorSubcoreMesh(core_axis_name="c", subcore_axis_name="s")

    @pl.kernel(
        mesh=mesh,
        out_type=jax.ShapeDtypeStruct((N, D), jnp.float32),
        scratch_types=[
            pltpu.VMEM((CHUNK,), jnp.int32),
            pltpu.VMEM((NPW,), jnp.int32),
            pltpu.VMEM((CHUNK, D), jnp.float32),
            pltpu.VMEM((CHUNK, D), jnp.float32),
            pltpu.SemaphoreType.DMA,
            pltpu.SemaphoreType.DMA,
        ],
        compiler_params=pltpu.CompilerParams(use_tc_tiling_on_sc=False),
    )
    def body(ids_hbm, pat_hbm, table_hbm, out_hbm,
             pat_v, idx_v, rows0_v, rows1_v, sem_g, sem_out):
        wid = lax.axis_index("s") * NC + lax.axis_index("c")
        wbase = wid * NPW
        pltpu.sync_copy(pat_hbm, pat_v)
        pltpu.sync_copy(ids_hbm.at[pl.ds(wbase, NPW)], idx_v)

        @pl.loop(0, CPW)
        def _add_chunk(c):
            @pl.loop(0, CHUNK // L)
            def _add(i):
                dst = pl.ds(c * CHUNK + i * L, L)
                r = idx_v[dst] + pat_v[pl.ds(i * L, L)]
                idx_v[dst] = (r & -512) | ((r & 127) << 2) | ((r >> 7) & 3)

        def fire_drain_store(c, rows_v):
            handles = []
            for j in range(NG):
                sl = pl.ds(c * CHUNK + j * GRAN, GRAN)
                handles.append(pltpu.make_async_copy(
                    table_hbm.at[idx_v.at[sl]],
                    rows_v.at[pl.ds(j * GRAN, GRAN), :], sem_g))
            for h in handles:
                h.start()
            for h in handles:
                h.wait()
            st = pltpu.make_async_copy(
                rows_v, out_hbm.at[pl.ds(wbase + c * CHUNK, CHUNK)], sem_out)
            st.start()
            return st

        def wait_store(rows_v):
            pltpu.make_async_copy(
                out_hbm.at[pl.ds(wbase, CHUNK)], rows_v, sem_out).wait()

        fire_drain_store(0, rows0_v)
        fire_drain_store(1, rows1_v)

        @pl.loop(2, CPW, step=2)
        def _chunk(c):
            wait_store(rows0_v)
            fire_drain_store(c, rows0_v)
            wait_store(rows1_v)
            fire_drain_store(c + 1, rows1_v)

        wait_store(rows0_v)
        wait_store(rows1_v)

    return body(flat_ids, pattern, table)


def kernel(input_ids, offsets, table):
    tbl_lin = _tc_table_format(table.T)
    tbl_rm = tbl_lin.reshape(VPAD, D)
    flat_ids = input_ids.reshape(N)
    pattern = jnp.tile(offsets, CHUNK // H)
    out = _sc_gather(flat_ids, pattern, tbl_rm)
    return out.reshape(B, H, D)


# speedup vs baseline: 14.3858x; 1.4514x over previous
import jax
import jax.numpy as jnp
from jax import lax
from jax.experimental import pallas as pl
from jax.experimental.pallas import tpu as pltpu
from jax.experimental.pallas import tpu_sc as plsc

B = 16384
H = 26
D = 32
N = B * H
V = 100000 * H
NC, NS, L = 2, 16, 16
NW = NC * NS
CHUNK = 1024
CPW = N // (NW * CHUNK)
NPW = CPW * CHUNK
GRAN = 128
NG = CHUNK // GRAN

TBLK = 8192
TGRID = (V + TBLK - 1) // TBLK
VPAD = TGRID * TBLK


def _tc_table_format(tbl_t):
    def body(i_ref, o_ref):
        x = i_ref[...]
        for q in range(TBLK // 512):
            sq = jnp.concatenate(
                [x[:, q * 512 + a * 128:q * 512 + (a + 1) * 128]
                 for a in range(4)], axis=0)
            o_ref[pl.ds(q * 128, 128), :] = sq.T

    return pl.pallas_call(
        body,
        grid=(TGRID,),
        in_specs=[pl.BlockSpec((D, TBLK), lambda i: (0, i))],
        out_specs=pl.BlockSpec((TBLK // 4, 128), lambda i: (i, 0)),
        out_shape=jax.ShapeDtypeStruct((VPAD * D // 128, 128), jnp.float32),
    )(tbl_t)


def _tc_out_format(flat):
    SLAB = B * D // 128

    def body(i_ref, o_ref):
        x = i_ref[...]
        for q in range(B // 512):
            t = x[q * 128:(q + 1) * 128, :].T
            for a in range(4):
                for dt in range(4):
                    o_ref[pl.ds((dt * 128 + 4 * q + a) * 8, 8), :] = (
                        t[32 * a + 8 * dt:32 * a + 8 * dt + 8, :])

    return pl.pallas_call(
        body,
        grid=(H,),
        in_specs=[pl.BlockSpec((SLAB, 128), lambda h: (h, 0))],
        out_specs=pl.BlockSpec((SLAB, 128), lambda h: (h, 0)),
        out_shape=jax.ShapeDtypeStruct((H * SLAB, 128), jnp.float32),
    )(flat)


def _sc_gather(ids_hm, offs_pad, table):
    mesh = plsc.VectorSubcoreMesh(core_axis_name="c", subcore_axis_name="s")

    @pl.kernel(
        mesh=mesh,
        out_type=jax.ShapeDtypeStruct((N, D), jnp.float32),
        scratch_types=[
            pltpu.VMEM((32,), jnp.int32),
            pltpu.VMEM((NPW,), jnp.int32),
            pltpu.VMEM((NPW,), jnp.int32),
            pltpu.VMEM((CHUNK, D), jnp.float32),
            pltpu.VMEM((CHUNK, D), jnp.float32),
            pltpu.SemaphoreType.DMA,
            pltpu.SemaphoreType.DMA,
        ],
        compiler_params=pltpu.CompilerParams(
            use_tc_tiling_on_sc=False, needs_layout_passes=False),
    )
    def body(ids_hbm, off_hbm, table_hbm, out_hbm,
             offs_v, ids_v, idx_v, rows0_v, rows1_v, sem_g, sem_out):
        wid = lax.axis_index("s") * NC + lax.axis_index("c")
        wbase = wid * NPW
        pltpu.sync_copy(off_hbm, offs_v)
        pltpu.sync_copy(ids_hbm.at[pl.ds(wbase, NPW)], ids_v)

        iot = jax.lax.iota(jnp.int32, L)
        pat = ((iot & 3) << 7) | (iot >> 2)

        @pl.loop(0, NPW // L)
        def _slots(tg):
            kl = tg * L
            src = ((kl & -512) + ((kl >> 2) & 127)) + pat
            raw = plsc.load_gather(ids_v, [src])
            h = (wbase + src) >> 14
            r = raw + plsc.load_gather(offs_v, [h])
            idx_v[pl.ds(kl, L)] = (r & -512) | ((r & 127) << 2) | ((r >> 7) & 3)

        def fire_drain_store(c, rows_v):
            handles = []
            for j in range(NG):
                sl = pl.ds(c * CHUNK + j * GRAN, GRAN)
                handles.append(pltpu.make_async_copy(
                    table_hbm.at[idx_v.at[sl]],
                    rows_v.at[pl.ds(j * GRAN, GRAN), :], sem_g))
            for hd in handles:
                hd.start()
            for hd in handles:
                hd.wait()
            st = pltpu.make_async_copy(
                rows_v, out_hbm.at[pl.ds(wbase + c * CHUNK, CHUNK)], sem_out)
            st.start()
            return st

        def wait_store(rows_v):
            pltpu.make_async_copy(
                out_hbm.at[pl.ds(wbase, CHUNK)], rows_v, sem_out).wait()

        fire_drain_store(0, rows0_v)
        fire_drain_store(1, rows1_v)

        @pl.loop(2, CPW - 1, step=2)
        def _chunk(c):
            wait_store(rows0_v)
            fire_drain_store(c, rows0_v)
            wait_store(rows1_v)
            fire_drain_store(c + 1, rows1_v)

        wait_store(rows0_v)
        fire_drain_store(CPW - 1, rows0_v)
        wait_store(rows1_v)
        wait_store(rows0_v)

    return body(ids_hm, offs_pad, table)


def kernel(input_ids, offsets, table):
    tbl_lin = _tc_table_format(table.T)
    tbl_rm = tbl_lin.reshape(VPAD, D)
    ids_hm = input_ids.T.reshape(N)
    offs_pad = jnp.pad(offsets, (0, 32 - H))
    rows = _sc_gather(ids_hm, offs_pad, tbl_rm)
    fmt = _tc_out_format(rows.reshape(N * D // 128, 128))
    return fmt.reshape(H, 4, B // 128, 8, 128).transpose(
        2, 4, 0, 1, 3).reshape(B, H, D)


# speedup vs baseline: 18.1124x; 1.2590x over previous
import jax
import jax.numpy as jnp
from jax import lax
from jax.experimental import pallas as pl
from jax.experimental.pallas import tpu as pltpu
from jax.experimental.pallas import tpu_sc as plsc

B = 16384
H = 26
D = 32
N = B * H
V = 100000 * H
NC, NS, L = 2, 16, 16
NW = NC * NS
CHUNK = 1024
CPW = N // (NW * CHUNK)
NPW = CPW * CHUNK
GRAN = 128
NG = CHUNK // GRAN

TBLK = 16384
TGRID = (V + TBLK - 1) // TBLK
VPAD = TGRID * TBLK


def _tc_table_format(tbl_t):
    def body(i_ref, o_ref):
        x = i_ref[...]
        for q in range(TBLK // 512):
            sq = jnp.concatenate(
                [x[:, q * 512 + a * 128:q * 512 + (a + 1) * 128]
                 for a in range(4)], axis=0)
            o_ref[pl.ds(q * 128, 128), :] = sq.T

    return pl.pallas_call(
        body,
        grid=(TGRID,),
        in_specs=[pl.BlockSpec((D, TBLK), lambda i: (0, i))],
        out_specs=pl.BlockSpec((TBLK // 4, 128), lambda i: (i, 0)),
        out_shape=jax.ShapeDtypeStruct((VPAD * D // 128, 128), jnp.float32),
    )(tbl_t)


def _tc_out_format(flat):
    SLAB = B * D // 128

    def body(i_ref, o_ref):
        x = i_ref[...]
        for q in range(B // 512):
            t = x[q * 128:(q + 1) * 128, :].T
            for a in range(4):
                for dt in range(4):
                    o_ref[pl.ds((dt * 128 + 4 * q + a) * 8, 8), :] = (
                        t[32 * a + 8 * dt:32 * a + 8 * dt + 8, :])

    return pl.pallas_call(
        body,
        grid=(H,),
        in_specs=[pl.BlockSpec((SLAB, 128), lambda h: (h, 0))],
        out_specs=pl.BlockSpec((SLAB, 128), lambda h: (h, 0)),
        out_shape=jax.ShapeDtypeStruct((H * SLAB, 128), jnp.float32),
    )(flat)


def _sc_gather(ids_hm, offs_pad, table):
    mesh = plsc.VectorSubcoreMesh(core_axis_name="c", subcore_axis_name="s")

    @pl.kernel(
        mesh=mesh,
        out_type=jax.ShapeDtypeStruct((N, D), jnp.float32),
        scratch_types=[
            pltpu.VMEM((32,), jnp.int32),
            pltpu.VMEM((NPW,), jnp.int32),
            pltpu.VMEM((NPW,), jnp.int32),
            pltpu.VMEM((CHUNK, D), jnp.float32),
            pltpu.VMEM((CHUNK, D), jnp.float32),
            pltpu.SemaphoreType.DMA,
            pltpu.SemaphoreType.DMA,
        ],
        compiler_params=pltpu.CompilerParams(
            use_tc_tiling_on_sc=False, needs_layout_passes=False),
    )
    def body(ids_hbm, off_hbm, table_hbm, out_hbm,
             offs_v, ids_v, idx_v, rows0_v, rows1_v, sem_g, sem_out):
        wid = lax.axis_index("s") * NC + lax.axis_index("c")
        wbase = wid * NPW
        pltpu.sync_copy(off_hbm, offs_v)
        pltpu.sync_copy(ids_hbm.at[pl.ds(wbase, NPW)], ids_v)

        iot = jax.lax.iota(jnp.int32, L)
        pat = ((iot & 3) << 7) | (iot >> 2)

        @pl.loop(0, NPW // L)
        def _slots(tg):
            kl = tg * L
            src = ((kl & -512) + ((kl >> 2) & 127)) + pat
            raw = plsc.load_gather(ids_v, [src])
            h = (wbase + src) >> 14
            r = raw + plsc.load_gather(offs_v, [h])
            idx_v[pl.ds(kl, L)] = (r & -512) | ((r & 127) << 2) | ((r >> 7) & 3)

        def fire_drain_store(c, rows_v):
            handles = []
            for j in range(NG):
                sl = pl.ds(c * CHUNK + j * GRAN, GRAN)
                handles.append(pltpu.make_async_copy(
                    table_hbm.at[idx_v.at[sl]],
                    rows_v.at[pl.ds(j * GRAN, GRAN), :], sem_g))
            for hd in handles:
                hd.start()
            for hd in handles:
                hd.wait()
            st = pltpu.make_async_copy(
                rows_v, out_hbm.at[pl.ds(wbase + c * CHUNK, CHUNK)], sem_out)
            st.start()
            return st

        def wait_store(rows_v):
            pltpu.make_async_copy(
                out_hbm.at[pl.ds(wbase, CHUNK)], rows_v, sem_out).wait()

        fire_drain_store(0, rows0_v)
        fire_drain_store(1, rows1_v)

        @pl.loop(2, CPW - 1, step=2)
        def _chunk(c):
            wait_store(rows0_v)
            fire_drain_store(c, rows0_v)
            wait_store(rows1_v)
            fire_drain_store(c + 1, rows1_v)

        wait_store(rows0_v)
        fire_drain_store(CPW - 1, rows0_v)
        wait_store(rows1_v)
        wait_store(rows0_v)

    return body(ids_hm, offs_pad, table)


def kernel(input_ids, offsets, table):
    tbl_lin = _tc_table_format(table.T)
    tbl_rm = tbl_lin.reshape(VPAD, D)
    ids_hm = input_ids.T.reshape(N)
    offs_pad = jnp.pad(offsets, (0, 32 - H))
    rows = _sc_gather(ids_hm, offs_pad, tbl_rm)
    fmt = _tc_out_format(rows.reshape(N * D // 128, 128))
    return fmt.reshape(H, 4, B // 128, 8, 128).transpose(
        2, 4, 0, 1, 3).reshape(B, H, D)
